# baseline (speedup 1.0000x reference)
import functools

import jax
import jax.numpy as jnp
from jax import lax
from jax.experimental import pallas as pl
from jax.experimental.pallas import tpu as pltpu

_TN = 256


def _body(cnt, NT, unknown_ref, known_ref, kfeat_ref, ufeat_ref, w1_ref,
          gamma_ref, beta_ref, wr_ref, br_ref, we_ref, be_ref, out_ref,
          h_scr, stat_scr):
    phase = pl.program_id(0)
    b = pl.program_id(1)
    t = pl.program_id(2)
    s = b * NT + t

    @pl.when(phase == 0)
    def _phase0():
        TN = unknown_ref.shape[1]
        M = known_ref.shape[2]
        u = unknown_ref[0]
        k = known_ref[0]
        d2 = ((u[:, 0:1] - k[0:1, :]) ** 2
              + (u[:, 1:2] - k[1:2, :]) ** 2
              + (u[:, 2:3] - k[2:3, :]) ** 2)

        iota = lax.broadcasted_iota(jnp.int32, (TN, M), 1).astype(jnp.float32)
        inf = jnp.float32(jnp.inf)
        fM = jnp.float32(M)

        v1 = jnp.min(d2, axis=1, keepdims=True)
        i1 = jnp.min(jnp.where(d2 == v1, iota, fM), axis=1, keepdims=True)
        d2b = jnp.where(iota == i1, inf, d2)
        v2 = jnp.min(d2b, axis=1, keepdims=True)
        i2 = jnp.min(jnp.where(d2b == v2, iota, fM), axis=1, keepdims=True)
        d2c = jnp.where(iota == i2, inf, d2b)
        v3 = jnp.min(d2c, axis=1, keepdims=True)
        i3 = jnp.min(jnp.where(d2c == v3, iota, fM), axis=1, keepdims=True)

        r1 = 1.0 / (jnp.sqrt(jnp.maximum(v1, 0.0)) + 1e-8)
        r2 = 1.0 / (jnp.sqrt(jnp.maximum(v2, 0.0)) + 1e-8)
        r3 = 1.0 / (jnp.sqrt(jnp.maximum(v3, 0.0)) + 1e-8)
        norm = r1 + r2 + r3
        zero = jnp.float32(0.0)
        oh = (jnp.where(iota == i1, r1 / norm, zero)
              + jnp.where(iota == i2, r2 / norm, zero)
              + jnp.where(iota == i3, r3 / norm, zero))

        kf = kfeat_ref[0]
        interp = lax.dot_general(kf, oh, (((1,), (1,)), ((), ())),
                                 preferred_element_type=jnp.float32)
        uf = ufeat_ref[0]
        W1 = w1_ref[...]
        C2 = kf.shape[0]
        h = (lax.dot_general(W1[:, :C2], interp, (((1,), (0,)), ((), ())),
                             preferred_element_type=jnp.float32)
             + lax.dot_general(W1[:, C2:], uf, (((1,), (0,)), ((), ())),
                               preferred_element_type=jnp.float32))
        h_scr[s] = h

        @pl.when(s == 0)
        def _():
            stat_scr[...] = jnp.zeros_like(stat_scr)

        stat_scr[:, 0:1] += jnp.sum(h, axis=1, keepdims=True)
        stat_scr[:, 1:2] += jnp.sum(h * h, axis=1, keepdims=True)

    @pl.when(phase == 1)
    def _phase1():
        h = h_scr[s]
        mean = stat_scr[:, 0:1] / cnt
        var = stat_scr[:, 1:2] / cnt - mean * mean
        hn = (h - mean) / jnp.sqrt(var + 1e-5) * gamma_ref[...] + beta_ref[...]
        hn = jnp.maximum(hn, 0.0)
        sq = lax.dot_general(wr_ref[...], hn, (((1,), (0,)), ((), ())),
                             preferred_element_type=jnp.float32) + br_ref[...]
        sq = sq * jax.nn.sigmoid(sq)
        e = lax.dot_general(we_ref[...], sq, (((1,), (0,)), ((), ())),
                            preferred_element_type=jnp.float32) + be_ref[...]
        out_ref[0] = jax.nn.sigmoid(e) * hn


def kernel(unknown, known, unknow_feats, known_feats, W1, gamma, beta, Wr, br,
           We, be):
    B, N, _ = unknown.shape
    M = known.shape[1]
    C2 = known_feats.shape[1]
    C1 = unknow_feats.shape[1]
    COUT, CIN = W1.shape
    NSQ = Wr.shape[0]
    TN = _TN
    NT = N // TN
    cnt = float(B * N)

    known_t = jnp.pad(jnp.transpose(known, (0, 2, 1)),
                      ((0, 0), (0, 5), (0, 0)))
    NSQP = 8
    wr_p = jnp.pad(Wr, ((0, NSQP - NSQ), (0, 0)))
    br_p = jnp.pad(br, (0, NSQP - NSQ)).reshape(NSQP, 1)
    we_p = jnp.pad(We, ((0, 0), (0, NSQP - NSQ)))
    gamma_c = gamma.reshape(COUT, 1)
    beta_c = beta.reshape(COUT, 1)
    be_c = be.reshape(COUT, 1)

    out = pl.pallas_call(
        functools.partial(_body, cnt, NT),
        grid=(2, B, NT),
        in_specs=[
            pl.BlockSpec((1, TN, 3), lambda p, b, t: (b, t, 0)),
            pl.BlockSpec((1, 8, M), lambda p, b, t: (b, 0, 0)),
            pl.BlockSpec((1, C2, M), lambda p, b, t: (b, 0, 0)),
            pl.BlockSpec((1, C1, TN), lambda p, b, t: (b, 0, t)),
            pl.BlockSpec((COUT, CIN), lambda p, b, t: (0, 0)),
            pl.BlockSpec((COUT, 1), lambda p, b, t: (0, 0)),
            pl.BlockSpec((COUT, 1), lambda p, b, t: (0, 0)),
            pl.BlockSpec((NSQP, COUT), lambda p, b, t: (0, 0)),
            pl.BlockSpec((NSQP, 1), lambda p, b, t: (0, 0)),
            pl.BlockSpec((COUT, NSQP), lambda p, b, t: (0, 0)),
            pl.BlockSpec((COUT, 1), lambda p, b, t: (0, 0)),
        ],
        out_specs=pl.BlockSpec((1, COUT, TN),
                               lambda p, b, t: (b * p, 0, t * p)),
        out_shape=jax.ShapeDtypeStruct((B, COUT, N), jnp.float32),
        scratch_shapes=[
            pltpu.VMEM((B * NT, COUT, TN), jnp.float32),
            pltpu.VMEM((COUT, 2), jnp.float32),
        ],
    )(unknown, known_t, known_feats, unknow_feats, W1,
      gamma_c, beta_c, wr_p, br_p, we_p, be_c)
    return out


# speedup vs baseline: 1.2164x; 1.2164x over previous
import functools

import jax
import jax.numpy as jnp
from jax import lax
from jax.experimental import pallas as pl

_TN = 256


def _pass1_body(unknown_ref, known_ref, kfeat_ref, ufeat_ref, w1_ref,
                hpre_ref, stat_ref):
    TN = unknown_ref.shape[1]
    M = known_ref.shape[2]
    u = unknown_ref[0]
    k = known_ref[0]
    d2 = ((u[:, 0:1] - k[0:1, :]) ** 2
          + (u[:, 1:2] - k[1:2, :]) ** 2
          + (u[:, 2:3] - k[2:3, :]) ** 2)

    inf = jnp.float32(jnp.inf)
    v1 = jnp.min(d2, axis=1, keepdims=True)
    m1 = d2 == v1
    d2b = jnp.where(m1, inf, d2)
    v2 = jnp.min(d2b, axis=1, keepdims=True)
    m2 = d2b == v2
    d2c = jnp.where(m2, inf, d2b)
    v3 = jnp.min(d2c, axis=1, keepdims=True)
    m3 = d2c == v3

    r1 = 1.0 / (jnp.sqrt(jnp.maximum(v1, 0.0)) + 1e-8)
    r2 = 1.0 / (jnp.sqrt(jnp.maximum(v2, 0.0)) + 1e-8)
    r3 = 1.0 / (jnp.sqrt(jnp.maximum(v3, 0.0)) + 1e-8)
    norm = r1 + r2 + r3
    zero = jnp.float32(0.0)
    oh = (jnp.where(m1, r1 / norm, zero)
          + jnp.where(m2, r2 / norm, zero)
          + jnp.where(m3, r3 / norm, zero))

    kf = kfeat_ref[0]
    interp = lax.dot_general(kf, oh, (((1,), (1,)), ((), ())),
                             preferred_element_type=jnp.float32)
    uf = ufeat_ref[0]
    W1 = w1_ref[...]
    C2 = kf.shape[0]
    h = (lax.dot_general(W1[:, :C2], interp, (((1,), (0,)), ((), ())),
                         preferred_element_type=jnp.float32)
         + lax.dot_general(W1[:, C2:], uf, (((1,), (0,)), ((), ())),
                           preferred_element_type=jnp.float32))
    hpre_ref[0] = h

    first = (pl.program_id(0) == 0) & (pl.program_id(1) == 0)

    @pl.when(first)
    def _():
        stat_ref[...] = jnp.zeros_like(stat_ref)

    stat_ref[:, 0:1] += jnp.sum(h, axis=1, keepdims=True)
    stat_ref[:, 1:2] += jnp.sum(h * h, axis=1, keepdims=True)


def _pass2_body(cnt, hpre_ref, stat_ref, gamma_ref, beta_ref, wr_ref, br_ref,
                we_ref, be_ref, out_ref):
    h = hpre_ref[0]
    mean = stat_ref[:, 0:1] / cnt
    var = stat_ref[:, 1:2] / cnt - mean * mean
    hn = (h - mean) / jnp.sqrt(var + 1e-5) * gamma_ref[...] + beta_ref[...]
    hn = jnp.maximum(hn, 0.0)
    s = lax.dot_general(wr_ref[...], hn, (((1,), (0,)), ((), ())),
                        preferred_element_type=jnp.float32) + br_ref[...]
    s = s * jax.nn.sigmoid(s)
    e = lax.dot_general(we_ref[...], s, (((1,), (0,)), ((), ())),
                        preferred_element_type=jnp.float32) + be_ref[...]
    out_ref[0] = jax.nn.sigmoid(e) * hn


def kernel(unknown, known, unknow_feats, known_feats, W1, gamma, beta, Wr, br,
           We, be):
    B, N, _ = unknown.shape
    M = known.shape[1]
    C2 = known_feats.shape[1]
    C1 = unknow_feats.shape[1]
    COUT, CIN = W1.shape
    NSQ = Wr.shape[0]
    TN = _TN
    NT = N // TN
    cnt = float(B * N)

    known_t = jnp.pad(jnp.transpose(known, (0, 2, 1)),
                      ((0, 0), (0, 5), (0, 0)))

    hpre, stat = pl.pallas_call(
        _pass1_body,
        grid=(B, NT),
        in_specs=[
            pl.BlockSpec((1, TN, 3), lambda b, t: (b, t, 0)),
            pl.BlockSpec((1, 8, M), lambda b, t: (b, 0, 0)),
            pl.BlockSpec((1, C2, M), lambda b, t: (b, 0, 0)),
            pl.BlockSpec((1, C1, TN), lambda b, t: (b, 0, t)),
            pl.BlockSpec((COUT, CIN), lambda b, t: (0, 0)),
        ],
        out_specs=[
            pl.BlockSpec((1, COUT, TN), lambda b, t: (b, 0, t)),
            pl.BlockSpec((COUT, 2), lambda b, t: (0, 0)),
        ],
        out_shape=[
            jax.ShapeDtypeStruct((B, COUT, N), jnp.float32),
            jax.ShapeDtypeStruct((COUT, 2), jnp.float32),
        ],
    )(unknown, known_t, known_feats, unknow_feats, W1)

    NSQP = 8
    wr_p = jnp.pad(Wr, ((0, NSQP - NSQ), (0, 0)))
    br_p = jnp.pad(br, (0, NSQP - NSQ)).reshape(NSQP, 1)
    we_p = jnp.pad(We, ((0, 0), (0, NSQP - NSQ)))
    gamma_c = gamma.reshape(COUT, 1)
    beta_c = beta.reshape(COUT, 1)
    be_c = be.reshape(COUT, 1)

    out = pl.pallas_call(
        functools.partial(_pass2_body, cnt),
        grid=(B, NT),
        in_specs=[
            pl.BlockSpec((1, COUT, TN), lambda b, t: (b, 0, t)),
            pl.BlockSpec((COUT, 2), lambda b, t: (0, 0)),
            pl.BlockSpec((COUT, 1), lambda b, t: (0, 0)),
            pl.BlockSpec((COUT, 1), lambda b, t: (0, 0)),
            pl.BlockSpec((NSQP, COUT), lambda b, t: (0, 0)),
            pl.BlockSpec((NSQP, 1), lambda b, t: (0, 0)),
            pl.BlockSpec((COUT, NSQP), lambda b, t: (0, 0)),
            pl.BlockSpec((COUT, 1), lambda b, t: (0, 0)),
        ],
        out_specs=pl.BlockSpec((1, COUT, TN), lambda b, t: (b, 0, t)),
        out_shape=jax.ShapeDtypeStruct((B, COUT, N), jnp.float32),
    )(hpre, stat, gamma_c, beta_c, wr_p, br_p, we_p, be_c)
    return out


# speedup vs baseline: 1.2621x; 1.0375x over previous
import functools

import jax
import jax.numpy as jnp
from jax import lax
from jax.experimental import pallas as pl

_TN = 256


def _pass1_body(unknown_ref, known_ref, kfeat_ref, ufeat_ref, w1_ref,
                hpre_ref, stat_ref):
    TN = unknown_ref.shape[1]
    M = known_ref.shape[2]
    u = unknown_ref[0]
    k = known_ref[0]
    d2 = ((u[:, 0:1] - k[0:1, :]) ** 2
          + (u[:, 1:2] - k[1:2, :]) ** 2
          + (u[:, 2:3] - k[2:3, :]) ** 2)

    inf = jnp.float32(jnp.inf)
    v1 = jnp.min(d2, axis=1, keepdims=True)
    m1 = d2 == v1
    d2b = jnp.where(m1, inf, d2)
    v2 = jnp.min(d2b, axis=1, keepdims=True)
    m2 = d2b == v2
    d2c = jnp.where(m2, inf, d2b)
    v3 = jnp.min(d2c, axis=1, keepdims=True)
    m3 = d2c == v3

    r1 = 1.0 / (jnp.sqrt(jnp.maximum(v1, 0.0)) + 1e-8)
    r2 = 1.0 / (jnp.sqrt(jnp.maximum(v2, 0.0)) + 1e-8)
    r3 = 1.0 / (jnp.sqrt(jnp.maximum(v3, 0.0)) + 1e-8)
    norm = r1 + r2 + r3
    zero = jnp.float32(0.0)
    oh = jnp.where(m1, r1 / norm,
                   jnp.where(m2, r2 / norm,
                             jnp.where(m3, r3 / norm, zero)))

    kf = kfeat_ref[0]
    interp = lax.dot_general(kf, oh, (((1,), (1,)), ((), ())),
                             preferred_element_type=jnp.float32)
    uf = ufeat_ref[0]
    W1 = w1_ref[...]
    C2 = kf.shape[0]
    h = (lax.dot_general(W1[:, :C2], interp, (((1,), (0,)), ((), ())),
                         preferred_element_type=jnp.float32)
         + lax.dot_general(W1[:, C2:], uf, (((1,), (0,)), ((), ())),
                           preferred_element_type=jnp.float32))
    hpre_ref[0] = h

    first = (pl.program_id(0) == 0) & (pl.program_id(1) == 0)

    @pl.when(first)
    def _():
        stat_ref[...] = jnp.zeros_like(stat_ref)

    stat_ref[:, 0:1] += jnp.sum(h, axis=1, keepdims=True)
    stat_ref[:, 1:2] += jnp.sum(h * h, axis=1, keepdims=True)


def _pass2_body(cnt, hpre_ref, stat_ref, gamma_ref, beta_ref, wr_ref, br_ref,
                we_ref, be_ref, out_ref):
    h = hpre_ref[0]
    mean = stat_ref[:, 0:1] / cnt
    var = stat_ref[:, 1:2] / cnt - mean * mean
    hn = (h - mean) / jnp.sqrt(var + 1e-5) * gamma_ref[...] + beta_ref[...]
    hn = jnp.maximum(hn, 0.0)
    s = lax.dot_general(wr_ref[...], hn, (((1,), (0,)), ((), ())),
                        preferred_element_type=jnp.float32) + br_ref[...]
    s = s * jax.nn.sigmoid(s)
    e = lax.dot_general(we_ref[...], s, (((1,), (0,)), ((), ())),
                        preferred_element_type=jnp.float32) + be_ref[...]
    out_ref[0] = jax.nn.sigmoid(e) * hn


def kernel(unknown, known, unknow_feats, known_feats, W1, gamma, beta, Wr, br,
           We, be):
    B, N, _ = unknown.shape
    M = known.shape[1]
    C2 = known_feats.shape[1]
    C1 = unknow_feats.shape[1]
    COUT, CIN = W1.shape
    NSQ = Wr.shape[0]
    TN = _TN
    NT = N // TN
    cnt = float(B * N)

    known_t = jnp.pad(jnp.transpose(known, (0, 2, 1)),
                      ((0, 0), (0, 5), (0, 0)))

    hpre, stat = pl.pallas_call(
        _pass1_body,
        grid=(B, NT),
        in_specs=[
            pl.BlockSpec((1, TN, 3), lambda b, t: (b, t, 0)),
            pl.BlockSpec((1, 8, M), lambda b, t: (b, 0, 0)),
            pl.BlockSpec((1, C2, M), lambda b, t: (b, 0, 0)),
            pl.BlockSpec((1, C1, TN), lambda b, t: (b, 0, t)),
            pl.BlockSpec((COUT, CIN), lambda b, t: (0, 0)),
        ],
        out_specs=[
            pl.BlockSpec((1, COUT, TN), lambda b, t: (b, 0, t)),
            pl.BlockSpec((COUT, 2), lambda b, t: (0, 0)),
        ],
        out_shape=[
            jax.ShapeDtypeStruct((B, COUT, N), jnp.float32),
            jax.ShapeDtypeStruct((COUT, 2), jnp.float32),
        ],
    )(unknown, known_t, known_feats, unknow_feats, W1)

    NSQP = 8
    wr_p = jnp.pad(Wr, ((0, NSQP - NSQ), (0, 0)))
    br_p = jnp.pad(br, (0, NSQP - NSQ)).reshape(NSQP, 1)
    we_p = jnp.pad(We, ((0, 0), (0, NSQP - NSQ)))
    gamma_c = gamma.reshape(COUT, 1)
    beta_c = beta.reshape(COUT, 1)
    be_c = be.reshape(COUT, 1)

    out = pl.pallas_call(
        functools.partial(_pass2_body, cnt),
        grid=(B, NT),
        in_specs=[
            pl.BlockSpec((1, COUT, TN), lambda b, t: (b, 0, t)),
            pl.BlockSpec((COUT, 2), lambda b, t: (0, 0)),
            pl.BlockSpec((COUT, 1), lambda b, t: (0, 0)),
            pl.BlockSpec((COUT, 1), lambda b, t: (0, 0)),
            pl.BlockSpec((NSQP, COUT), lambda b, t: (0, 0)),
            pl.BlockSpec((NSQP, 1), lambda b, t: (0, 0)),
            pl.BlockSpec((COUT, NSQP), lambda b, t: (0, 0)),
            pl.BlockSpec((COUT, 1), lambda b, t: (0, 0)),
        ],
        out_specs=pl.BlockSpec((1, COUT, TN), lambda b, t: (b, 0, t)),
        out_shape=jax.ShapeDtypeStruct((B, COUT, N), jnp.float32),
    )(hpre, stat, gamma_c, beta_c, wr_p, br_p, we_p, be_c)
    return out


# speedup vs baseline: 1.5101x; 1.1966x over previous
import functools

import jax
import jax.numpy as jnp
from jax import lax
from jax.experimental import pallas as pl

_TN = 512


def _pass1_body(unknown_ref, known_ref, kfeat_ref, ufeat_ref, w1_ref,
                hpre_ref, stat_ref):
    TN = unknown_ref.shape[1]
    M = known_ref.shape[2]
    u = unknown_ref[0]
    k = known_ref[0]
    d2 = ((u[:, 0:1] - k[0:1, :]) ** 2
          + (u[:, 1:2] - k[1:2, :]) ** 2
          + (u[:, 2:3] - k[2:3, :]) ** 2)

    inf = jnp.float32(jnp.inf)
    v1 = jnp.min(d2, axis=1, keepdims=True)
    m1 = d2 == v1
    d2b = jnp.where(m1, inf, d2)
    v2 = jnp.min(d2b, axis=1, keepdims=True)
    m2 = d2b == v2
    d2c = jnp.where(m2, inf, d2b)
    v3 = jnp.min(d2c, axis=1, keepdims=True)
    m3 = d2c == v3

    r1 = 1.0 / (jnp.sqrt(jnp.maximum(v1, 0.0)) + 1e-8)
    r2 = 1.0 / (jnp.sqrt(jnp.maximum(v2, 0.0)) + 1e-8)
    r3 = 1.0 / (jnp.sqrt(jnp.maximum(v3, 0.0)) + 1e-8)
    norm = r1 + r2 + r3
    zero = jnp.float32(0.0)
    oh = jnp.where(m1, r1 / norm,
                   jnp.where(m2, r2 / norm,
                             jnp.where(m3, r3 / norm, zero)))

    kf = kfeat_ref[0]
    interp = lax.dot_general(kf, oh, (((1,), (1,)), ((), ())),
                             preferred_element_type=jnp.float32)
    uf = ufeat_ref[0]
    W1 = w1_ref[...]
    C2 = kf.shape[0]
    h = (lax.dot_general(W1[:, :C2], interp, (((1,), (0,)), ((), ())),
                         preferred_element_type=jnp.float32)
         + lax.dot_general(W1[:, C2:], uf, (((1,), (0,)), ((), ())),
                           preferred_element_type=jnp.float32))
    hpre_ref[0] = h

    first = (pl.program_id(0) == 0) & (pl.program_id(1) == 0)

    @pl.when(first)
    def _():
        stat_ref[...] = jnp.zeros_like(stat_ref)

    stat_ref[:, 0:1] += jnp.sum(h, axis=1, keepdims=True)
    stat_ref[:, 1:2] += jnp.sum(h * h, axis=1, keepdims=True)


def _pass2_body(cnt, hpre_ref, stat_ref, gamma_ref, beta_ref, wr_ref, br_ref,
                we_ref, be_ref, out_ref):
    h = hpre_ref[0]
    mean = stat_ref[:, 0:1] / cnt
    var = stat_ref[:, 1:2] / cnt - mean * mean
    hn = (h - mean) / jnp.sqrt(var + 1e-5) * gamma_ref[...] + beta_ref[...]
    hn = jnp.maximum(hn, 0.0)
    s = lax.dot_general(wr_ref[...], hn, (((1,), (0,)), ((), ())),
                        preferred_element_type=jnp.float32) + br_ref[...]
    s = s * jax.nn.sigmoid(s)
    e = lax.dot_general(we_ref[...], s, (((1,), (0,)), ((), ())),
                        preferred_element_type=jnp.float32) + be_ref[...]
    out_ref[0] = jax.nn.sigmoid(e) * hn


def kernel(unknown, known, unknow_feats, known_feats, W1, gamma, beta, Wr, br,
           We, be):
    B, N, _ = unknown.shape
    M = known.shape[1]
    C2 = known_feats.shape[1]
    C1 = unknow_feats.shape[1]
    COUT, CIN = W1.shape
    NSQ = Wr.shape[0]
    TN = _TN
    NT = N // TN
    cnt = float(B * N)

    known_t = jnp.pad(jnp.transpose(known, (0, 2, 1)),
                      ((0, 0), (0, 5), (0, 0)))

    hpre, stat = pl.pallas_call(
        _pass1_body,
        grid=(B, NT),
        in_specs=[
            pl.BlockSpec((1, TN, 3), lambda b, t: (b, t, 0)),
            pl.BlockSpec((1, 8, M), lambda b, t: (b, 0, 0)),
            pl.BlockSpec((1, C2, M), lambda b, t: (b, 0, 0)),
            pl.BlockSpec((1, C1, TN), lambda b, t: (b, 0, t)),
            pl.BlockSpec((COUT, CIN), lambda b, t: (0, 0)),
        ],
        out_specs=[
            pl.BlockSpec((1, COUT, TN), lambda b, t: (b, 0, t)),
            pl.BlockSpec((COUT, 2), lambda b, t: (0, 0)),
        ],
        out_shape=[
            jax.ShapeDtypeStruct((B, COUT, N), jnp.float32),
            jax.ShapeDtypeStruct((COUT, 2), jnp.float32),
        ],
    )(unknown, known_t, known_feats, unknow_feats, W1)

    NSQP = 8
    wr_p = jnp.pad(Wr, ((0, NSQP - NSQ), (0, 0)))
    br_p = jnp.pad(br, (0, NSQP - NSQ)).reshape(NSQP, 1)
    we_p = jnp.pad(We, ((0, 0), (0, NSQP - NSQ)))
    gamma_c = gamma.reshape(COUT, 1)
    beta_c = beta.reshape(COUT, 1)
    be_c = be.reshape(COUT, 1)

    out = pl.pallas_call(
        functools.partial(_pass2_body, cnt),
        grid=(B, NT),
        in_specs=[
            pl.BlockSpec((1, COUT, TN), lambda b, t: (b, 0, t)),
            pl.BlockSpec((COUT, 2), lambda b, t: (0, 0)),
            pl.BlockSpec((COUT, 1), lambda b, t: (0, 0)),
            pl.BlockSpec((COUT, 1), lambda b, t: (0, 0)),
            pl.BlockSpec((NSQP, COUT), lambda b, t: (0, 0)),
            pl.BlockSpec((NSQP, 1), lambda b, t: (0, 0)),
            pl.BlockSpec((COUT, NSQP), lambda b, t: (0, 0)),
            pl.BlockSpec((COUT, 1), lambda b, t: (0, 0)),
        ],
        out_specs=pl.BlockSpec((1, COUT, TN), lambda b, t: (b, 0, t)),
        out_shape=jax.ShapeDtypeStruct((B, COUT, N), jnp.float32),
    )(hpre, stat, gamma_c, beta_c, wr_p, br_p, we_p, be_c)
    return out


# speedup vs baseline: 1.7030x; 1.1277x over previous
import functools

import jax
import jax.numpy as jnp
from jax import lax
from jax.experimental import pallas as pl

_TN = 1024


def _pass1_body(unknown_ref, known_ref, kfeat_ref, ufeat_ref, w1_ref,
                hpre_ref, stat_ref):
    TN = unknown_ref.shape[1]
    M = known_ref.shape[2]
    u = unknown_ref[0]
    k = known_ref[0]
    d2 = ((u[:, 0:1] - k[0:1, :]) ** 2
          + (u[:, 1:2] - k[1:2, :]) ** 2
          + (u[:, 2:3] - k[2:3, :]) ** 2)

    inf = jnp.float32(jnp.inf)
    v1 = jnp.min(d2, axis=1, keepdims=True)
    m1 = d2 == v1
    d2b = jnp.where(m1, inf, d2)
    v2 = jnp.min(d2b, axis=1, keepdims=True)
    m2 = d2b == v2
    d2c = jnp.where(m2, inf, d2b)
    v3 = jnp.min(d2c, axis=1, keepdims=True)
    m3 = d2c == v3

    r1 = 1.0 / (jnp.sqrt(jnp.maximum(v1, 0.0)) + 1e-8)
    r2 = 1.0 / (jnp.sqrt(jnp.maximum(v2, 0.0)) + 1e-8)
    r3 = 1.0 / (jnp.sqrt(jnp.maximum(v3, 0.0)) + 1e-8)
    norm = r1 + r2 + r3
    zero = jnp.float32(0.0)
    oh = jnp.where(m1, r1 / norm,
                   jnp.where(m2, r2 / norm,
                             jnp.where(m3, r3 / norm, zero)))

    kf = kfeat_ref[0]
    interp = lax.dot_general(kf, oh, (((1,), (1,)), ((), ())),
                             preferred_element_type=jnp.float32)
    uf = ufeat_ref[0]
    W1 = w1_ref[...]
    C2 = kf.shape[0]
    h = (lax.dot_general(W1[:, :C2], interp, (((1,), (0,)), ((), ())),
                         preferred_element_type=jnp.float32)
         + lax.dot_general(W1[:, C2:], uf, (((1,), (0,)), ((), ())),
                           preferred_element_type=jnp.float32))
    hpre_ref[0] = h

    first = (pl.program_id(0) == 0) & (pl.program_id(1) == 0)

    @pl.when(first)
    def _():
        stat_ref[...] = jnp.zeros_like(stat_ref)

    stat_ref[:, 0:1] += jnp.sum(h, axis=1, keepdims=True)
    stat_ref[:, 1:2] += jnp.sum(h * h, axis=1, keepdims=True)


def _pass2_body(cnt, hpre_ref, stat_ref, gamma_ref, beta_ref, wr_ref, br_ref,
                we_ref, be_ref, out_ref):
    h = hpre_ref[0]
    mean = stat_ref[:, 0:1] / cnt
    var = stat_ref[:, 1:2] / cnt - mean * mean
    hn = (h - mean) / jnp.sqrt(var + 1e-5) * gamma_ref[...] + beta_ref[...]
    hn = jnp.maximum(hn, 0.0)
    s = lax.dot_general(wr_ref[...], hn, (((1,), (0,)), ((), ())),
                        preferred_element_type=jnp.float32) + br_ref[...]
    s = s * jax.nn.sigmoid(s)
    e = lax.dot_general(we_ref[...], s, (((1,), (0,)), ((), ())),
                        preferred_element_type=jnp.float32) + be_ref[...]
    out_ref[0] = jax.nn.sigmoid(e) * hn


def kernel(unknown, known, unknow_feats, known_feats, W1, gamma, beta, Wr, br,
           We, be):
    B, N, _ = unknown.shape
    M = known.shape[1]
    C2 = known_feats.shape[1]
    C1 = unknow_feats.shape[1]
    COUT, CIN = W1.shape
    NSQ = Wr.shape[0]
    TN = _TN
    NT = N // TN
    cnt = float(B * N)

    known_t = jnp.pad(jnp.transpose(known, (0, 2, 1)),
                      ((0, 0), (0, 5), (0, 0)))

    hpre, stat = pl.pallas_call(
        _pass1_body,
        grid=(B, NT),
        in_specs=[
            pl.BlockSpec((1, TN, 3), lambda b, t: (b, t, 0)),
            pl.BlockSpec((1, 8, M), lambda b, t: (b, 0, 0)),
            pl.BlockSpec((1, C2, M), lambda b, t: (b, 0, 0)),
            pl.BlockSpec((1, C1, TN), lambda b, t: (b, 0, t)),
            pl.BlockSpec((COUT, CIN), lambda b, t: (0, 0)),
        ],
        out_specs=[
            pl.BlockSpec((1, COUT, TN), lambda b, t: (b, 0, t)),
            pl.BlockSpec((COUT, 2), lambda b, t: (0, 0)),
        ],
        out_shape=[
            jax.ShapeDtypeStruct((B, COUT, N), jnp.float32),
            jax.ShapeDtypeStruct((COUT, 2), jnp.float32),
        ],
    )(unknown, known_t, known_feats, unknow_feats, W1)

    NSQP = 8
    wr_p = jnp.pad(Wr, ((0, NSQP - NSQ), (0, 0)))
    br_p = jnp.pad(br, (0, NSQP - NSQ)).reshape(NSQP, 1)
    we_p = jnp.pad(We, ((0, 0), (0, NSQP - NSQ)))
    gamma_c = gamma.reshape(COUT, 1)
    beta_c = beta.reshape(COUT, 1)
    be_c = be.reshape(COUT, 1)

    out = pl.pallas_call(
        functools.partial(_pass2_body, cnt),
        grid=(B, NT),
        in_specs=[
            pl.BlockSpec((1, COUT, TN), lambda b, t: (b, 0, t)),
            pl.BlockSpec((COUT, 2), lambda b, t: (0, 0)),
            pl.BlockSpec((COUT, 1), lambda b, t: (0, 0)),
            pl.BlockSpec((COUT, 1), lambda b, t: (0, 0)),
            pl.BlockSpec((NSQP, COUT), lambda b, t: (0, 0)),
            pl.BlockSpec((NSQP, 1), lambda b, t: (0, 0)),
            pl.BlockSpec((COUT, NSQP), lambda b, t: (0, 0)),
            pl.BlockSpec((COUT, 1), lambda b, t: (0, 0)),
        ],
        out_specs=pl.BlockSpec((1, COUT, TN), lambda b, t: (b, 0, t)),
        out_shape=jax.ShapeDtypeStruct((B, COUT, N), jnp.float32),
    )(hpre, stat, gamma_c, beta_c, wr_p, br_p, we_p, be_c)
    return out


# speedup vs baseline: 1.7897x; 1.0509x over previous
import functools

import jax
import jax.numpy as jnp
from jax import lax
from jax.experimental import pallas as pl

_TN = 2048


def _pass1_body(unknown_ref, known_ref, kfeat_ref, ufeat_ref, w1_ref,
                hpre_ref, stat_ref):
    TN = unknown_ref.shape[1]
    M = known_ref.shape[2]
    u = unknown_ref[0]
    k = known_ref[0]
    d2 = ((u[:, 0:1] - k[0:1, :]) ** 2
          + (u[:, 1:2] - k[1:2, :]) ** 2
          + (u[:, 2:3] - k[2:3, :]) ** 2)

    inf = jnp.float32(jnp.inf)
    v1 = jnp.min(d2, axis=1, keepdims=True)
    m1 = d2 == v1
    d2b = jnp.where(m1, inf, d2)
    v2 = jnp.min(d2b, axis=1, keepdims=True)
    m2 = d2b == v2
    d2c = jnp.where(m2, inf, d2b)
    v3 = jnp.min(d2c, axis=1, keepdims=True)
    m3 = d2c == v3

    r1 = 1.0 / (jnp.sqrt(jnp.maximum(v1, 0.0)) + 1e-8)
    r2 = 1.0 / (jnp.sqrt(jnp.maximum(v2, 0.0)) + 1e-8)
    r3 = 1.0 / (jnp.sqrt(jnp.maximum(v3, 0.0)) + 1e-8)
    norm = r1 + r2 + r3
    zero = jnp.float32(0.0)
    oh = jnp.where(m1, r1 / norm,
                   jnp.where(m2, r2 / norm,
                             jnp.where(m3, r3 / norm, zero)))

    kf = kfeat_ref[0]
    interp = lax.dot_general(kf, oh, (((1,), (1,)), ((), ())),
                             preferred_element_type=jnp.float32)
    uf = ufeat_ref[0]
    W1 = w1_ref[...]
    C2 = kf.shape[0]
    h = (lax.dot_general(W1[:, :C2], interp, (((1,), (0,)), ((), ())),
                         preferred_element_type=jnp.float32)
         + lax.dot_general(W1[:, C2:], uf, (((1,), (0,)), ((), ())),
                           preferred_element_type=jnp.float32))
    hpre_ref[0] = h

    first = (pl.program_id(0) == 0) & (pl.program_id(1) == 0)

    @pl.when(first)
    def _():
        stat_ref[...] = jnp.zeros_like(stat_ref)

    stat_ref[:, 0:1] += jnp.sum(h, axis=1, keepdims=True)
    stat_ref[:, 1:2] += jnp.sum(h * h, axis=1, keepdims=True)


def _pass2_body(cnt, hpre_ref, stat_ref, gamma_ref, beta_ref, wr_ref, br_ref,
                we_ref, be_ref, out_ref):
    h = hpre_ref[0]
    mean = stat_ref[:, 0:1] / cnt
    var = stat_ref[:, 1:2] / cnt - mean * mean
    hn = (h - mean) / jnp.sqrt(var + 1e-5) * gamma_ref[...] + beta_ref[...]
    hn = jnp.maximum(hn, 0.0)
    s = lax.dot_general(wr_ref[...], hn, (((1,), (0,)), ((), ())),
                        preferred_element_type=jnp.float32) + br_ref[...]
    s = s * jax.nn.sigmoid(s)
    e = lax.dot_general(we_ref[...], s, (((1,), (0,)), ((), ())),
                        preferred_element_type=jnp.float32) + be_ref[...]
    out_ref[0] = jax.nn.sigmoid(e) * hn


def kernel(unknown, known, unknow_feats, known_feats, W1, gamma, beta, Wr, br,
           We, be):
    B, N, _ = unknown.shape
    M = known.shape[1]
    C2 = known_feats.shape[1]
    C1 = unknow_feats.shape[1]
    COUT, CIN = W1.shape
    NSQ = Wr.shape[0]
    TN = _TN
    NT = N // TN
    cnt = float(B * N)

    known_t = jnp.pad(jnp.transpose(known, (0, 2, 1)),
                      ((0, 0), (0, 5), (0, 0)))

    hpre, stat = pl.pallas_call(
        _pass1_body,
        grid=(B, NT),
        in_specs=[
            pl.BlockSpec((1, TN, 3), lambda b, t: (b, t, 0)),
            pl.BlockSpec((1, 8, M), lambda b, t: (b, 0, 0)),
            pl.BlockSpec((1, C2, M), lambda b, t: (b, 0, 0)),
            pl.BlockSpec((1, C1, TN), lambda b, t: (b, 0, t)),
            pl.BlockSpec((COUT, CIN), lambda b, t: (0, 0)),
        ],
        out_specs=[
            pl.BlockSpec((1, COUT, TN), lambda b, t: (b, 0, t)),
            pl.BlockSpec((COUT, 2), lambda b, t: (0, 0)),
        ],
        out_shape=[
            jax.ShapeDtypeStruct((B, COUT, N), jnp.float32),
            jax.ShapeDtypeStruct((COUT, 2), jnp.float32),
        ],
    )(unknown, known_t, known_feats, unknow_feats, W1)

    NSQP = 8
    wr_p = jnp.pad(Wr, ((0, NSQP - NSQ), (0, 0)))
    br_p = jnp.pad(br, (0, NSQP - NSQ)).reshape(NSQP, 1)
    we_p = jnp.pad(We, ((0, 0), (0, NSQP - NSQ)))
    gamma_c = gamma.reshape(COUT, 1)
    beta_c = beta.reshape(COUT, 1)
    be_c = be.reshape(COUT, 1)

    out = pl.pallas_call(
        functools.partial(_pass2_body, cnt),
        grid=(B, NT),
        in_specs=[
            pl.BlockSpec((1, COUT, TN), lambda b, t: (b, 0, t)),
            pl.BlockSpec((COUT, 2), lambda b, t: (0, 0)),
            pl.BlockSpec((COUT, 1), lambda b, t: (0, 0)),
            pl.BlockSpec((COUT, 1), lambda b, t: (0, 0)),
            pl.BlockSpec((NSQP, COUT), lambda b, t: (0, 0)),
            pl.BlockSpec((NSQP, 1), lambda b, t: (0, 0)),
            pl.BlockSpec((COUT, NSQP), lambda b, t: (0, 0)),
            pl.BlockSpec((COUT, 1), lambda b, t: (0, 0)),
        ],
        out_specs=pl.BlockSpec((1, COUT, TN), lambda b, t: (b, 0, t)),
        out_shape=jax.ShapeDtypeStruct((B, COUT, N), jnp.float32),
    )(hpre, stat, gamma_c, beta_c, wr_p, br_p, we_p, be_c)
    return out


# speedup vs baseline: 1.8241x; 1.0192x over previous
import functools

import jax
import jax.numpy as jnp
from jax import lax
from jax.experimental import pallas as pl

_TN = 4096


def _pass1_body(unknown_ref, known_ref, kfeat_ref, ufeat_ref, w1_ref,
                hpre_ref, stat_ref):
    TN = unknown_ref.shape[1]
    M = known_ref.shape[2]
    u = unknown_ref[0]
    k = known_ref[0]
    d2 = ((u[:, 0:1] - k[0:1, :]) ** 2
          + (u[:, 1:2] - k[1:2, :]) ** 2
          + (u[:, 2:3] - k[2:3, :]) ** 2)

    inf = jnp.float32(jnp.inf)
    v1 = jnp.min(d2, axis=1, keepdims=True)
    m1 = d2 == v1
    d2b = jnp.where(m1, inf, d2)
    v2 = jnp.min(d2b, axis=1, keepdims=True)
    m2 = d2b == v2
    d2c = jnp.where(m2, inf, d2b)
    v3 = jnp.min(d2c, axis=1, keepdims=True)
    m3 = d2c == v3

    r1 = 1.0 / (jnp.sqrt(jnp.maximum(v1, 0.0)) + 1e-8)
    r2 = 1.0 / (jnp.sqrt(jnp.maximum(v2, 0.0)) + 1e-8)
    r3 = 1.0 / (jnp.sqrt(jnp.maximum(v3, 0.0)) + 1e-8)
    norm = r1 + r2 + r3
    zero = jnp.float32(0.0)
    oh = jnp.where(m1, r1 / norm,
                   jnp.where(m2, r2 / norm,
                             jnp.where(m3, r3 / norm, zero)))

    kf = kfeat_ref[0]
    interp = lax.dot_general(kf, oh, (((1,), (1,)), ((), ())),
                             preferred_element_type=jnp.float32)
    uf = ufeat_ref[0]
    W1 = w1_ref[...]
    C2 = kf.shape[0]
    h = (lax.dot_general(W1[:, :C2], interp, (((1,), (0,)), ((), ())),
                         preferred_element_type=jnp.float32)
         + lax.dot_general(W1[:, C2:], uf, (((1,), (0,)), ((), ())),
                           preferred_element_type=jnp.float32))
    hpre_ref[0] = h

    first = (pl.program_id(0) == 0) & (pl.program_id(1) == 0)

    @pl.when(first)
    def _():
        stat_ref[...] = jnp.zeros_like(stat_ref)

    stat_ref[:, 0:1] += jnp.sum(h, axis=1, keepdims=True)
    stat_ref[:, 1:2] += jnp.sum(h * h, axis=1, keepdims=True)


def _pass2_body(cnt, hpre_ref, stat_ref, gamma_ref, beta_ref, wr_ref, br_ref,
                we_ref, be_ref, out_ref):
    h = hpre_ref[0]
    mean = stat_ref[:, 0:1] / cnt
    var = stat_ref[:, 1:2] / cnt - mean * mean
    hn = (h - mean) / jnp.sqrt(var + 1e-5) * gamma_ref[...] + beta_ref[...]
    hn = jnp.maximum(hn, 0.0)
    s = lax.dot_general(wr_ref[...], hn, (((1,), (0,)), ((), ())),
                        preferred_element_type=jnp.float32) + br_ref[...]
    s = s * jax.nn.sigmoid(s)
    e = lax.dot_general(we_ref[...], s, (((1,), (0,)), ((), ())),
                        preferred_element_type=jnp.float32) + be_ref[...]
    out_ref[0] = jax.nn.sigmoid(e) * hn


def kernel(unknown, known, unknow_feats, known_feats, W1, gamma, beta, Wr, br,
           We, be):
    B, N, _ = unknown.shape
    M = known.shape[1]
    C2 = known_feats.shape[1]
    C1 = unknow_feats.shape[1]
    COUT, CIN = W1.shape
    NSQ = Wr.shape[0]
    TN = _TN
    NT = N // TN
    cnt = float(B * N)

    known_t = jnp.pad(jnp.transpose(known, (0, 2, 1)),
                      ((0, 0), (0, 5), (0, 0)))

    hpre, stat = pl.pallas_call(
        _pass1_body,
        grid=(B, NT),
        in_specs=[
            pl.BlockSpec((1, TN, 3), lambda b, t: (b, t, 0)),
            pl.BlockSpec((1, 8, M), lambda b, t: (b, 0, 0)),
            pl.BlockSpec((1, C2, M), lambda b, t: (b, 0, 0)),
            pl.BlockSpec((1, C1, TN), lambda b, t: (b, 0, t)),
            pl.BlockSpec((COUT, CIN), lambda b, t: (0, 0)),
        ],
        out_specs=[
            pl.BlockSpec((1, COUT, TN), lambda b, t: (b, 0, t)),
            pl.BlockSpec((COUT, 2), lambda b, t: (0, 0)),
        ],
        out_shape=[
            jax.ShapeDtypeStruct((B, COUT, N), jnp.float32),
            jax.ShapeDtypeStruct((COUT, 2), jnp.float32),
        ],
    )(unknown, known_t, known_feats, unknow_feats, W1)

    NSQP = 8
    wr_p = jnp.pad(Wr, ((0, NSQP - NSQ), (0, 0)))
    br_p = jnp.pad(br, (0, NSQP - NSQ)).reshape(NSQP, 1)
    we_p = jnp.pad(We, ((0, 0), (0, NSQP - NSQ)))
    gamma_c = gamma.reshape(COUT, 1)
    beta_c = beta.reshape(COUT, 1)
    be_c = be.reshape(COUT, 1)

    out = pl.pallas_call(
        functools.partial(_pass2_body, cnt),
        grid=(B, NT),
        in_specs=[
            pl.BlockSpec((1, COUT, TN), lambda b, t: (b, 0, t)),
            pl.BlockSpec((COUT, 2), lambda b, t: (0, 0)),
            pl.BlockSpec((COUT, 1), lambda b, t: (0, 0)),
            pl.BlockSpec((COUT, 1), lambda b, t: (0, 0)),
            pl.BlockSpec((NSQP, COUT), lambda b, t: (0, 0)),
            pl.BlockSpec((NSQP, 1), lambda b, t: (0, 0)),
            pl.BlockSpec((COUT, NSQP), lambda b, t: (0, 0)),
            pl.BlockSpec((COUT, 1), lambda b, t: (0, 0)),
        ],
        out_specs=pl.BlockSpec((1, COUT, TN), lambda b, t: (b, 0, t)),
        out_shape=jax.ShapeDtypeStruct((B, COUT, N), jnp.float32),
    )(hpre, stat, gamma_c, beta_c, wr_p, br_p, we_p, be_c)
    return out
